# baseline (speedup 1.0000x reference)
import functools

import jax
import jax.numpy as jnp
import numpy as np
from jax.experimental import pallas as pl
from jax.experimental.pallas import tpu as pltpu

_TB = 8192


def _halfword_weights(H):
    W = H // 32
    h = np.arange(H)
    row = ((h >> 4) & 1) * W + (h >> 5)
    wh = np.zeros((2 * W, H), np.float32)
    wh[row, h] = 2.0 ** (h & 15)
    return jnp.asarray(wh, jnp.bfloat16)


def _prep_kernel(theta_ref, wh_ref, sl_ref, gl_ref, p_ref, c_ref, b_ref,
                 *, max_slip, max_guess, W):
    nb = jnp.where(theta_ref[...] <= 0.0, 1.0, 0.0).astype(jnp.bfloat16)
    hw = jax.lax.dot_general(wh_ref[...], nb, (((1,), (1,)), ((), ())),
                             preferred_element_type=jnp.float32)
    wi = hw.astype(jnp.int32)
    p_ref[...] = jnp.bitwise_or(wi[:W, :], jax.lax.shift_left(wi[W:, :], 16))
    slip = jax.nn.sigmoid(sl_ref[...]) * max_slip
    guess = jax.nn.sigmoid(gl_ref[...]) * max_guess
    lg = jnp.log(guess)
    c_ref[...] = jnp.log(1.0 - slip) - lg
    b_ref[...] = lg


def _main_kernel(planes_ref, ctab_ref, btab_ref, wh_ref, uid_ref, iid_ref,
                 know_ref, out_ref, *, W, SU, SI, TS):
    knb = know_ref[0].astype(jnp.bfloat16)
    knb3 = knb.reshape(TS, 128, knb.shape[1])
    hw = jax.lax.dot_general(wh_ref[...], knb3, (((1,), (2,)), ((), ())),
                             preferred_element_type=jnp.float32)
    wi = hw.astype(jnp.int32)
    kw = [jnp.bitwise_or(wi[k], jax.lax.shift_left(wi[W + k], 16))
          for k in range(W)]

    uid = uid_ref[0]
    ul = jnp.bitwise_and(uid, 127)
    us = jax.lax.shift_right_logical(uid, 7)
    g = [jnp.zeros_like(kw[0]) for _ in range(W)]
    for sv in range(SU):
        m = us == sv
        for k in range(W):
            cand = jnp.take_along_axis(planes_ref[k, sv], ul, axis=1)
            g[k] = jnp.where(m, cand, g[k])

    scount = jax.lax.population_count(jnp.bitwise_and(g[0], kw[0]))
    for k in range(1, W):
        scount = scount + jax.lax.population_count(jnp.bitwise_and(g[k], kw[k]))
    n = jnp.exp2(-scount.astype(jnp.float32))

    iid = iid_ref[0]
    il = jnp.bitwise_and(iid, 127)
    isv = jax.lax.shift_right_logical(iid, 7)
    cacc = jnp.zeros(n.shape, jnp.float32)
    bacc = jnp.zeros(n.shape, jnp.float32)
    for sv in range(SI):
        m = isv == sv
        cacc = jnp.where(m, jnp.take_along_axis(ctab_ref[sv], il, axis=1), cacc)
        bacc = jnp.where(m, jnp.take_along_axis(btab_ref[sv], il, axis=1), bacc)

    out_ref[0] = jnp.exp(bacc + n * cacc)


def kernel(user, item, knowledge, theta_table, slip_table, guess_table):
    max_slip = 0.4
    max_guess = 0.4
    B = user.shape[0]
    U, H = theta_table.shape
    I = slip_table.shape[0]
    W = H // 32
    SU = U // 128
    SI = I // 128

    wh = _halfword_weights(H)

    prep = pl.pallas_call(
        functools.partial(_prep_kernel, max_slip=max_slip, max_guess=max_guess,
                          W=W),
        out_shape=(jax.ShapeDtypeStruct((W, U), jnp.int32),
                   jax.ShapeDtypeStruct((SI, 128), jnp.float32),
                   jax.ShapeDtypeStruct((SI, 128), jnp.float32)),
    )
    P, c2, b2 = prep(theta_table,
                     wh,
                     slip_table.reshape(SI, 128),
                     guess_table.reshape(SI, 128))

    TB = _TB
    TS = TB // 128
    NB = pl.cdiv(B, TB)
    Bp = NB * TB
    uid = jnp.asarray(user, jnp.int32)
    iid = jnp.asarray(item, jnp.int32)
    know = jnp.asarray(knowledge, jnp.float32)
    if Bp != B:
        uid = jnp.pad(uid, (0, Bp - B))
        iid = jnp.pad(iid, (0, Bp - B))
        know = jnp.pad(know, ((0, Bp - B), (0, 0)))

    planes = jnp.broadcast_to(P.reshape(W, SU, 1, 128), (W, SU, TS, 128))
    ctab = jnp.broadcast_to(c2[:, None, :], (SI, TS, 128))
    btab = jnp.broadcast_to(b2[:, None, :], (SI, TS, 128))

    out3 = pl.pallas_call(
        functools.partial(_main_kernel, W=W, SU=SU, SI=SI, TS=TS),
        out_shape=jax.ShapeDtypeStruct((NB, TS, 128), jnp.float32),
        grid_spec=pltpu.PrefetchScalarGridSpec(
            num_scalar_prefetch=0,
            grid=(NB,),
            in_specs=[
                pl.BlockSpec((W, SU, TS, 128), lambda i: (0, 0, 0, 0)),
                pl.BlockSpec((SI, TS, 128), lambda i: (0, 0, 0)),
                pl.BlockSpec((SI, TS, 128), lambda i: (0, 0, 0)),
                pl.BlockSpec((2 * W, H), lambda i: (0, 0)),
                pl.BlockSpec((1, TS, 128), lambda i: (i, 0, 0)),
                pl.BlockSpec((1, TS, 128), lambda i: (i, 0, 0)),
                pl.BlockSpec((1, TB, H), lambda i: (i, 0, 0)),
            ],
            out_specs=pl.BlockSpec((1, TS, 128), lambda i: (i, 0, 0)),
        ),
        compiler_params=pltpu.CompilerParams(
            dimension_semantics=("parallel",)),
    )(planes, ctab, btab, wh,
      uid.reshape(NB, TS, 128), iid.reshape(NB, TS, 128),
      know.reshape(NB, TB, H))
    return out3.reshape(Bp)[:B]


# speedup vs baseline: 1.8975x; 1.8975x over previous
import functools

import jax
import jax.numpy as jnp
import numpy as np
from jax.experimental import pallas as pl
from jax.experimental.pallas import tpu as pltpu

_TB = 8192


def _halfword_weights(H):
    W = H // 32
    h = np.arange(H)
    row = ((h >> 4) & 1) * W + (h >> 5)
    wh = np.zeros((2 * W, H), np.float32)
    wh[row, h] = 2.0 ** (h & 15)
    return jnp.asarray(wh, jnp.bfloat16)


def _prep_kernel(theta_ref, wh_ref, sl_ref, gl_ref, p_ref, c_ref, b_ref,
                 *, max_slip, max_guess, W):
    nb = jnp.where(theta_ref[...] <= 0.0, 1.0, 0.0).astype(jnp.bfloat16)
    hw = jax.lax.dot_general(wh_ref[...], nb, (((1,), (1,)), ((), ())),
                             preferred_element_type=jnp.float32)
    wi = hw.astype(jnp.int32)
    p_ref[...] = jnp.bitwise_or(wi[:W, :], jax.lax.shift_left(wi[W:, :], 16))
    slip = jax.nn.sigmoid(sl_ref[...]) * max_slip
    guess = jax.nn.sigmoid(gl_ref[...]) * max_guess
    lg = jnp.log(guess)
    c_ref[...] = jnp.log(1.0 - slip) - lg
    b_ref[...] = lg


def _main_kernel(planes_ref, ctab_ref, btab_ref, wh_ref, uid_ref, iid_ref,
                 know_ref, out_ref, *, W, SU, SI, TS):
    for c in range(TS // 8):
        rows = slice(c * 8, c * 8 + 8)
        knb = know_ref[0, c * 1024:(c + 1) * 1024, :].astype(jnp.bfloat16)
        knb3 = knb.reshape(8, 128, knb.shape[1])
        hw = jax.lax.dot_general(wh_ref[...], knb3, (((1,), (2,)), ((), ())),
                                 preferred_element_type=jnp.float32)
        wi = hw.astype(jnp.int32)
        kw = [jnp.bitwise_or(wi[k], jax.lax.shift_left(wi[W + k], 16))
              for k in range(W)]

        uid = uid_ref[0, rows, :]
        ul = jnp.bitwise_and(uid, 127)
        us = jax.lax.shift_right_logical(uid, 7)
        g = [jnp.zeros_like(kw[0]) for _ in range(W)]
        for sv in range(SU):
            m = us == sv
            for k in range(W):
                cand = jnp.take_along_axis(planes_ref[k, sv], ul, axis=1)
                g[k] = jnp.where(m, cand, g[k])

        scount = jax.lax.population_count(jnp.bitwise_and(g[0], kw[0]))
        for k in range(1, W):
            scount = scount + jax.lax.population_count(
                jnp.bitwise_and(g[k], kw[k]))
        n = jnp.exp2(-scount.astype(jnp.float32))

        iid = iid_ref[0, rows, :]
        il = jnp.bitwise_and(iid, 127)
        isv = jax.lax.shift_right_logical(iid, 7)
        cacc = jnp.zeros(n.shape, jnp.float32)
        bacc = jnp.zeros(n.shape, jnp.float32)
        for sv in range(SI):
            m = isv == sv
            cacc = jnp.where(m, jnp.take_along_axis(ctab_ref[sv], il, axis=1),
                             cacc)
            bacc = jnp.where(m, jnp.take_along_axis(btab_ref[sv], il, axis=1),
                             bacc)

        out_ref[0, rows, :] = jnp.exp(bacc + n * cacc)


def kernel(user, item, knowledge, theta_table, slip_table, guess_table):
    max_slip = 0.4
    max_guess = 0.4
    B = user.shape[0]
    U, H = theta_table.shape
    I = slip_table.shape[0]
    W = H // 32
    SU = U // 128
    SI = I // 128

    wh = _halfword_weights(H)

    prep = pl.pallas_call(
        functools.partial(_prep_kernel, max_slip=max_slip, max_guess=max_guess,
                          W=W),
        out_shape=(jax.ShapeDtypeStruct((W, U), jnp.int32),
                   jax.ShapeDtypeStruct((SI, 128), jnp.float32),
                   jax.ShapeDtypeStruct((SI, 128), jnp.float32)),
    )
    P, c2, b2 = prep(theta_table,
                     wh,
                     slip_table.reshape(SI, 128),
                     guess_table.reshape(SI, 128))

    TB = _TB
    TS = TB // 128
    NB = pl.cdiv(B, TB)
    Bp = NB * TB
    uid = jnp.asarray(user, jnp.int32)
    iid = jnp.asarray(item, jnp.int32)
    know = jnp.asarray(knowledge, jnp.float32)
    if Bp != B:
        uid = jnp.pad(uid, (0, Bp - B))
        iid = jnp.pad(iid, (0, Bp - B))
        know = jnp.pad(know, ((0, Bp - B), (0, 0)))

    planes = jnp.broadcast_to(P.reshape(W, SU, 1, 128), (W, SU, 8, 128))
    ctab = jnp.broadcast_to(c2[:, None, :], (SI, 8, 128))
    btab = jnp.broadcast_to(b2[:, None, :], (SI, 8, 128))

    out3 = pl.pallas_call(
        functools.partial(_main_kernel, W=W, SU=SU, SI=SI, TS=TS),
        out_shape=jax.ShapeDtypeStruct((NB, TS, 128), jnp.float32),
        grid_spec=pltpu.PrefetchScalarGridSpec(
            num_scalar_prefetch=0,
            grid=(NB,),
            in_specs=[
                pl.BlockSpec((W, SU, 8, 128), lambda i: (0, 0, 0, 0)),
                pl.BlockSpec((SI, 8, 128), lambda i: (0, 0, 0)),
                pl.BlockSpec((SI, 8, 128), lambda i: (0, 0, 0)),
                pl.BlockSpec((2 * W, H), lambda i: (0, 0)),
                pl.BlockSpec((1, TS, 128), lambda i: (i, 0, 0)),
                pl.BlockSpec((1, TS, 128), lambda i: (i, 0, 0)),
                pl.BlockSpec((1, TB, H), lambda i: (i, 0, 0)),
            ],
            out_specs=pl.BlockSpec((1, TS, 128), lambda i: (i, 0, 0)),
        ),
        compiler_params=pltpu.CompilerParams(
            dimension_semantics=("parallel",)),
    )(planes, ctab, btab, wh,
      uid.reshape(NB, TS, 128), iid.reshape(NB, TS, 128),
      know.reshape(NB, TB, H))
    return out3.reshape(Bp)[:B]


# speedup vs baseline: 2.1133x; 1.1137x over previous
import functools

import jax
import jax.numpy as jnp
import numpy as np
from jax.experimental import pallas as pl
from jax.experimental.pallas import tpu as pltpu

_TB = 16384


def _halfword_weights(H):
    W = H // 32
    h = np.arange(H)
    row = ((h >> 4) & 1) * W + (h >> 5)
    wh = np.zeros((2 * W, H), np.float32)
    wh[row, h] = 2.0 ** (h & 15)
    return jnp.asarray(wh, jnp.bfloat16)


def _prep_kernel(theta_ref, wh_ref, sl_ref, gl_ref, p_ref, c_ref, b_ref,
                 *, max_slip, max_guess, W):
    nb = jnp.where(theta_ref[...] <= 0.0, 1.0, 0.0).astype(jnp.bfloat16)
    hw = jax.lax.dot_general(wh_ref[...], nb, (((1,), (1,)), ((), ())),
                             preferred_element_type=jnp.float32)
    wi = hw.astype(jnp.int32)
    p_ref[...] = jnp.bitwise_or(wi[:W, :], jax.lax.shift_left(wi[W:, :], 16))
    slip = jax.nn.sigmoid(sl_ref[...]) * max_slip
    guess = jax.nn.sigmoid(gl_ref[...]) * max_guess
    lg = jnp.log(guess)
    c_ref[...] = jnp.log(1.0 - slip) - lg
    b_ref[...] = lg


def _main_kernel(planes_ref, ctab_ref, btab_ref, wh_ref, uid_ref, iid_ref,
                 know_ref, out_ref, *, W, SU, SI, TS):
    for c in range(TS // 8):
        rows = slice(c * 8, c * 8 + 8)
        knb = know_ref[0, c * 1024:(c + 1) * 1024, :].astype(jnp.bfloat16)
        knb3 = knb.reshape(8, 128, knb.shape[1])
        hw = jax.lax.dot_general(wh_ref[...], knb3, (((1,), (2,)), ((), ())),
                                 preferred_element_type=jnp.float32)
        wi = hw.astype(jnp.int32)
        kw = [jnp.bitwise_or(wi[k], jax.lax.shift_left(wi[W + k], 16))
              for k in range(W)]

        uid = uid_ref[0, rows, :]
        ul = jnp.bitwise_and(uid, 127)
        us = jax.lax.shift_right_logical(uid, 7)
        g = [jnp.zeros_like(kw[0]) for _ in range(W)]
        for sv in range(SU):
            m = us == sv
            for k in range(W):
                cand = jnp.take_along_axis(planes_ref[k, sv], ul, axis=1)
                g[k] = jnp.where(m, cand, g[k])

        scount = jax.lax.population_count(jnp.bitwise_and(g[0], kw[0]))
        for k in range(1, W):
            scount = scount + jax.lax.population_count(
                jnp.bitwise_and(g[k], kw[k]))
        n = jnp.exp2(-scount.astype(jnp.float32))

        iid = iid_ref[0, rows, :]
        il = jnp.bitwise_and(iid, 127)
        isv = jax.lax.shift_right_logical(iid, 7)
        cacc = jnp.zeros(n.shape, jnp.float32)
        bacc = jnp.zeros(n.shape, jnp.float32)
        for sv in range(SI):
            m = isv == sv
            cacc = jnp.where(m, jnp.take_along_axis(ctab_ref[sv], il, axis=1),
                             cacc)
            bacc = jnp.where(m, jnp.take_along_axis(btab_ref[sv], il, axis=1),
                             bacc)

        out_ref[0, rows, :] = jnp.exp(bacc + n * cacc)


def kernel(user, item, knowledge, theta_table, slip_table, guess_table):
    max_slip = 0.4
    max_guess = 0.4
    B = user.shape[0]
    U, H = theta_table.shape
    I = slip_table.shape[0]
    W = H // 32
    SU = U // 128
    SI = I // 128

    wh = _halfword_weights(H)

    prep = pl.pallas_call(
        functools.partial(_prep_kernel, max_slip=max_slip, max_guess=max_guess,
                          W=W),
        out_shape=(jax.ShapeDtypeStruct((W, U), jnp.int32),
                   jax.ShapeDtypeStruct((SI, 128), jnp.float32),
                   jax.ShapeDtypeStruct((SI, 128), jnp.float32)),
    )
    P, c2, b2 = prep(theta_table,
                     wh,
                     slip_table.reshape(SI, 128),
                     guess_table.reshape(SI, 128))

    TB = _TB
    TS = TB // 128
    NB = pl.cdiv(B, TB)
    Bp = NB * TB
    uid = jnp.asarray(user, jnp.int32)
    iid = jnp.asarray(item, jnp.int32)
    know = jnp.asarray(knowledge, jnp.float32)
    if Bp != B:
        uid = jnp.pad(uid, (0, Bp - B))
        iid = jnp.pad(iid, (0, Bp - B))
        know = jnp.pad(know, ((0, Bp - B), (0, 0)))

    planes = jnp.broadcast_to(P.reshape(W, SU, 1, 128), (W, SU, 8, 128))
    ctab = jnp.broadcast_to(c2[:, None, :], (SI, 8, 128))
    btab = jnp.broadcast_to(b2[:, None, :], (SI, 8, 128))

    out3 = pl.pallas_call(
        functools.partial(_main_kernel, W=W, SU=SU, SI=SI, TS=TS),
        out_shape=jax.ShapeDtypeStruct((NB, TS, 128), jnp.float32),
        grid_spec=pltpu.PrefetchScalarGridSpec(
            num_scalar_prefetch=0,
            grid=(NB,),
            in_specs=[
                pl.BlockSpec((W, SU, 8, 128), lambda i: (0, 0, 0, 0)),
                pl.BlockSpec((SI, 8, 128), lambda i: (0, 0, 0)),
                pl.BlockSpec((SI, 8, 128), lambda i: (0, 0, 0)),
                pl.BlockSpec((2 * W, H), lambda i: (0, 0)),
                pl.BlockSpec((1, TS, 128), lambda i: (i, 0, 0)),
                pl.BlockSpec((1, TS, 128), lambda i: (i, 0, 0)),
                pl.BlockSpec((1, TB, H), lambda i: (i, 0, 0)),
            ],
            out_specs=pl.BlockSpec((1, TS, 128), lambda i: (i, 0, 0)),
        ),
        compiler_params=pltpu.CompilerParams(
            dimension_semantics=("parallel",)),
    )(planes, ctab, btab, wh,
      uid.reshape(NB, TS, 128), iid.reshape(NB, TS, 128),
      know.reshape(NB, TB, H))
    return out3.reshape(Bp)[:B]


# speedup vs baseline: 2.2395x; 1.0597x over previous
import functools

import jax
import jax.numpy as jnp
import numpy as np
from jax.experimental import pallas as pl
from jax.experimental.pallas import tpu as pltpu

_TB = 16384


def _halfword_weights(H):
    W = H // 32
    h = np.arange(H)
    row = ((h >> 4) & 1) * W + (h >> 5)
    wh = np.zeros((2 * W, H), np.float32)
    wh[row, h] = 2.0 ** (h & 15)
    return jnp.asarray(wh, jnp.bfloat16)


def _prep_kernel(theta_ref, wh_ref, sl_ref, gl_ref, p_ref, q_ref,
                 *, max_slip, max_guess, W):
    nb = jnp.where(theta_ref[...] <= 0.0, 1.0, 0.0).astype(jnp.bfloat16)
    hw = jax.lax.dot_general(wh_ref[...], nb, (((1,), (1,)), ((), ())),
                             preferred_element_type=jnp.float32)
    wi = hw.astype(jnp.int32)
    p_ref[...] = jnp.bitwise_or(wi[:W, :], jax.lax.shift_left(wi[W:, :], 16))
    slip = jax.nn.sigmoid(sl_ref[...]) * max_slip
    guess = jax.nn.sigmoid(gl_ref[...]) * max_guess
    lg = jnp.log(guess)
    c = jnp.log(1.0 - slip) - lg
    ci = jax.lax.bitcast_convert_type(c, jnp.int32) + 0x8000
    bi = jax.lax.bitcast_convert_type(lg, jnp.int32) + 0x8000
    q_ref[...] = jnp.bitwise_or(jnp.bitwise_and(ci, jnp.int32(-65536)),
                                jax.lax.shift_right_logical(bi, 16))


def _main_kernel(planes_ref, qtab_ref, wh_ref, uid_ref, iid_ref,
                 know_ref, out_ref, *, W, SU, SI, TS):
    for c in range(TS // 8):
        rows = slice(c * 8, c * 8 + 8)
        knb = know_ref[0, c * 1024:(c + 1) * 1024, :].astype(jnp.bfloat16)
        knb3 = knb.reshape(8, 128, knb.shape[1])
        hw = jax.lax.dot_general(wh_ref[...], knb3, (((1,), (2,)), ((), ())),
                                 preferred_element_type=jnp.float32)
        wi = hw.astype(jnp.int32)
        kw = [jnp.bitwise_or(wi[k], jax.lax.shift_left(wi[W + k], 16))
              for k in range(W)]

        uid = uid_ref[0, rows, :]
        ul = jnp.bitwise_and(uid, 127)
        us = jax.lax.shift_right_logical(uid, 7)
        g = [jnp.zeros_like(kw[0]) for _ in range(W)]
        for sv in range(SU):
            m = us == sv
            for k in range(W):
                cand = jnp.take_along_axis(planes_ref[k, sv], ul, axis=1)
                g[k] = jnp.where(m, cand, g[k])

        scount = jax.lax.population_count(jnp.bitwise_and(g[0], kw[0]))
        for k in range(1, W):
            scount = scount + jax.lax.population_count(
                jnp.bitwise_and(g[k], kw[k]))
        n = jnp.exp2(-scount.astype(jnp.float32))

        iid = iid_ref[0, rows, :]
        il = jnp.bitwise_and(iid, 127)
        isv = jax.lax.shift_right_logical(iid, 7)
        qacc = jnp.zeros(n.shape, jnp.int32)
        for sv in range(SI):
            m = isv == sv
            qacc = jnp.where(m, jnp.take_along_axis(qtab_ref[sv], il, axis=1),
                             qacc)
        cc = jax.lax.bitcast_convert_type(
            jnp.bitwise_and(qacc, jnp.int32(-65536)), jnp.float32)
        bb = jax.lax.bitcast_convert_type(
            jax.lax.shift_left(qacc, 16), jnp.float32)

        out_ref[0, rows, :] = jnp.exp(bb + n * cc)


def kernel(user, item, knowledge, theta_table, slip_table, guess_table):
    max_slip = 0.4
    max_guess = 0.4
    B = user.shape[0]
    U, H = theta_table.shape
    I = slip_table.shape[0]
    W = H // 32
    SU = U // 128
    SI = I // 128

    wh = _halfword_weights(H)

    prep = pl.pallas_call(
        functools.partial(_prep_kernel, max_slip=max_slip, max_guess=max_guess,
                          W=W),
        out_shape=(jax.ShapeDtypeStruct((W, U), jnp.int32),
                   jax.ShapeDtypeStruct((SI, 128), jnp.int32)),
    )
    P, q2 = prep(theta_table,
                 wh,
                 slip_table.reshape(SI, 128),
                 guess_table.reshape(SI, 128))

    TB = _TB
    TS = TB // 128
    NB = pl.cdiv(B, TB)
    Bp = NB * TB
    uid = jnp.asarray(user, jnp.int32)
    iid = jnp.asarray(item, jnp.int32)
    know = jnp.asarray(knowledge, jnp.float32)
    if Bp != B:
        uid = jnp.pad(uid, (0, Bp - B))
        iid = jnp.pad(iid, (0, Bp - B))
        know = jnp.pad(know, ((0, Bp - B), (0, 0)))

    planes = jnp.broadcast_to(P.reshape(W, SU, 1, 128), (W, SU, 8, 128))
    qtab = jnp.broadcast_to(q2[:, None, :], (SI, 8, 128))

    out3 = pl.pallas_call(
        functools.partial(_main_kernel, W=W, SU=SU, SI=SI, TS=TS),
        out_shape=jax.ShapeDtypeStruct((NB, TS, 128), jnp.float32),
        grid_spec=pltpu.PrefetchScalarGridSpec(
            num_scalar_prefetch=0,
            grid=(NB,),
            in_specs=[
                pl.BlockSpec((W, SU, 8, 128), lambda i: (0, 0, 0, 0)),
                pl.BlockSpec((SI, 8, 128), lambda i: (0, 0, 0)),
                pl.BlockSpec((2 * W, H), lambda i: (0, 0)),
                pl.BlockSpec((1, TS, 128), lambda i: (i, 0, 0)),
                pl.BlockSpec((1, TS, 128), lambda i: (i, 0, 0)),
                pl.BlockSpec((1, TB, H), lambda i: (i, 0, 0)),
            ],
            out_specs=pl.BlockSpec((1, TS, 128), lambda i: (i, 0, 0)),
        ),
        compiler_params=pltpu.CompilerParams(
            dimension_semantics=("parallel",)),
    )(planes, qtab, wh,
      uid.reshape(NB, TS, 128), iid.reshape(NB, TS, 128),
      know.reshape(NB, TB, H))
    return out3.reshape(Bp)[:B]


# speedup vs baseline: 2.2431x; 1.0016x over previous
import functools

import jax
import jax.numpy as jnp
import numpy as np
from jax.experimental import pallas as pl
from jax.experimental.pallas import tpu as pltpu

_TB = 16384


def _halfword_weights(H):
    W = H // 32
    h = np.arange(H)
    row = ((h >> 4) & 1) * W + (h >> 5)
    wh = np.zeros((2 * W, H), np.float32)
    wh[row, h] = 2.0 ** (h & 15)
    return jnp.asarray(wh, jnp.bfloat16)


def _prep_kernel(theta_ref, wh_ref, sl_ref, gl_ref, p_ref, q_ref,
                 *, max_slip, max_guess, W):
    nb = jnp.where(theta_ref[...] <= 0.0, 1.0, 0.0).astype(jnp.bfloat16)
    hw = jax.lax.dot_general(wh_ref[...], nb, (((1,), (1,)), ((), ())),
                             preferred_element_type=jnp.float32)
    wi = hw.astype(jnp.int32)
    p_ref[...] = jnp.bitwise_or(wi[:W, :], jax.lax.shift_left(wi[W:, :], 16))
    slip = jax.nn.sigmoid(sl_ref[...]) * max_slip
    guess = jax.nn.sigmoid(gl_ref[...]) * max_guess
    lg = jnp.log(guess)
    c = jnp.log(1.0 - slip) - lg
    ci = jax.lax.bitcast_convert_type(c, jnp.int32) + 0x8000
    bi = jax.lax.bitcast_convert_type(lg, jnp.int32) + 0x8000
    q_ref[...] = jnp.bitwise_or(jnp.bitwise_and(ci, jnp.int32(-65536)),
                                jax.lax.shift_right_logical(bi, 16))


def _main_kernel(planes_ref, qtab_ref, wh_ref, uid_ref, iid_ref,
                 know_ref, out_ref, *, W, SU, SI, TS):
    for c in range(TS // 8):
        rows = slice(c * 8, c * 8 + 8)
        knb = know_ref[0, c * 1024:(c + 1) * 1024, :].astype(jnp.bfloat16)
        knb3 = knb.reshape(8, 128, knb.shape[1])
        hw = jax.lax.dot_general(wh_ref[...], knb3, (((1,), (2,)), ((), ())),
                                 preferred_element_type=jnp.float32)
        wi = jnp.round(hw).astype(jnp.int32)
        kw = [jnp.bitwise_or(wi[k], jax.lax.shift_left(wi[W + k], 16))
              for k in range(W)]

        uid = uid_ref[0, rows, :]
        ul = jnp.bitwise_and(uid, 127)
        us = jax.lax.shift_right_logical(uid, 7)
        nlev = SU.bit_length() - 1
        umask = [jnp.bitwise_and(us, 1 << t) != 0 for t in range(nlev)]
        g = []
        for k in range(W):
            lev = [jnp.take_along_axis(planes_ref[k, sv], ul, axis=1)
                   for sv in range(SU)]
            for t in range(nlev):
                lev = [jnp.where(umask[t], lev[2 * i + 1], lev[2 * i])
                       for i in range(len(lev) // 2)]
            g.append(lev[0])

        scount = jax.lax.population_count(jnp.bitwise_and(g[0], kw[0]))
        for k in range(1, W):
            scount = scount + jax.lax.population_count(
                jnp.bitwise_and(g[k], kw[k]))
        n = jnp.exp2(-scount.astype(jnp.float32))

        iid = iid_ref[0, rows, :]
        il = jnp.bitwise_and(iid, 127)
        isv = jax.lax.shift_right_logical(iid, 7)
        ilev = SI.bit_length() - 1
        imask = [jnp.bitwise_and(isv, 1 << t) != 0 for t in range(ilev)]
        lev = [jnp.take_along_axis(qtab_ref[sv], il, axis=1)
               for sv in range(SI)]
        for t in range(ilev):
            lev = [jnp.where(imask[t], lev[2 * i + 1], lev[2 * i])
                   for i in range(len(lev) // 2)]
        qacc = lev[0]
        cc = jax.lax.bitcast_convert_type(
            jnp.bitwise_and(qacc, jnp.int32(-65536)), jnp.float32)
        bb = jax.lax.bitcast_convert_type(
            jax.lax.shift_left(qacc, 16), jnp.float32)

        out_ref[0, rows, :] = jnp.exp(bb + n * cc)


def kernel(user, item, knowledge, theta_table, slip_table, guess_table):
    max_slip = 0.4
    max_guess = 0.4
    B = user.shape[0]
    U, H = theta_table.shape
    I = slip_table.shape[0]
    W = H // 32
    SU = U // 128
    SI = I // 128

    wh = _halfword_weights(H)

    prep = pl.pallas_call(
        functools.partial(_prep_kernel, max_slip=max_slip, max_guess=max_guess,
                          W=W),
        out_shape=(jax.ShapeDtypeStruct((W, U), jnp.int32),
                   jax.ShapeDtypeStruct((SI, 128), jnp.int32)),
    )
    P, q2 = prep(theta_table,
                 wh,
                 slip_table.reshape(SI, 128),
                 guess_table.reshape(SI, 128))

    TB = _TB
    TS = TB // 128
    NB = pl.cdiv(B, TB)
    Bp = NB * TB
    uid = jnp.asarray(user, jnp.int32)
    iid = jnp.asarray(item, jnp.int32)
    know = jnp.asarray(knowledge, jnp.float32)
    if Bp != B:
        uid = jnp.pad(uid, (0, Bp - B))
        iid = jnp.pad(iid, (0, Bp - B))
        know = jnp.pad(know, ((0, Bp - B), (0, 0)))

    planes = jnp.broadcast_to(P.reshape(W, SU, 1, 128), (W, SU, 8, 128))
    qtab = jnp.broadcast_to(q2[:, None, :], (SI, 8, 128))

    out3 = pl.pallas_call(
        functools.partial(_main_kernel, W=W, SU=SU, SI=SI, TS=TS),
        out_shape=jax.ShapeDtypeStruct((NB, TS, 128), jnp.float32),
        grid_spec=pltpu.PrefetchScalarGridSpec(
            num_scalar_prefetch=0,
            grid=(NB,),
            in_specs=[
                pl.BlockSpec((W, SU, 8, 128), lambda i: (0, 0, 0, 0)),
                pl.BlockSpec((SI, 8, 128), lambda i: (0, 0, 0)),
                pl.BlockSpec((2 * W, H), lambda i: (0, 0)),
                pl.BlockSpec((1, TS, 128), lambda i: (i, 0, 0)),
                pl.BlockSpec((1, TS, 128), lambda i: (i, 0, 0)),
                pl.BlockSpec((1, TB, H), lambda i: (i, 0, 0)),
            ],
            out_specs=pl.BlockSpec((1, TS, 128), lambda i: (i, 0, 0)),
        ),
        compiler_params=pltpu.CompilerParams(
            dimension_semantics=("parallel",)),
    )(planes, qtab, wh,
      uid.reshape(NB, TS, 128), iid.reshape(NB, TS, 128),
      know.reshape(NB, TB, H))
    return out3.reshape(Bp)[:B]
